# baseline (speedup 1.0000x reference)
import functools

import jax
import jax.numpy as jnp
from jax import lax
from jax.experimental import pallas as pl
from jax.experimental.pallas import tpu as pltpu
from jax.experimental.pallas import tpu_sc as plsc

F = 4
V = 100002
D = 64
DP = 128
NC = 2
NS = 16
NW = NC * NS
CH = 128


def _make_map_kernel(n_pad):
    wpf = NW // F
    per_w = F * n_pad // NW
    nch = per_w // CH
    mesh = plsc.VectorSubcoreMesh(core_axis_name="c", subcore_axis_name="s")

    @functools.partial(
        pl.kernel,
        out_type=jax.ShapeDtypeStruct((F * n_pad,), jnp.int32),
        mesh=mesh,
        scratch_types=[
            pltpu.VMEM((per_w,), jnp.int32),
            pltpu.VMEM((2, CH), jnp.int32),
            pltpu.VMEM((2, CH), jnp.int32),
            pltpu.SemaphoreType.DMA,
            pltpu.SemaphoreType.DMA,
        ],
    )
    def map_kernel(tok_hbm, map_hbm, fidx_hbm, tok_v, midx_v, val_v, sem0, sem1):
        c = lax.axis_index("c")
        s = lax.axis_index("s")
        wid = s * NC + c
        f = wid // wpf
        foff = f * V
        tbase = (wid - f * wpf) * per_w
        obase = f * n_pad + tbase
        sems = (sem0, sem1)
        pltpu.sync_copy(tok_hbm.at[pl.ds(tbase, per_w)], tok_v)

        def fill(j, buf):
            for i in range(CH // 16):
                midx_v[buf, pl.ds(i * 16, 16)] = (
                    tok_v[pl.ds(j * CH + i * 16, 16)] + foff)

        fill(0, 0)
        pltpu.async_copy(map_hbm.at[midx_v.at[0]], val_v.at[0], sems[0])

        def pair(i, carry):
            for p in range(2):
                j = 2 * i + p
                nxt = (p + 1) % 2

                @pl.when(j + 1 < nch)
                def _():
                    fill(j + 1, nxt)
                    pltpu.async_copy(
                        map_hbm.at[midx_v.at[nxt]], val_v.at[nxt], sems[nxt])

                pltpu.make_async_copy(
                    map_hbm.at[pl.ds(0, CH)], val_v.at[p], sems[p]).wait()
                pltpu.sync_copy(val_v.at[p], fidx_hbm.at[pl.ds(obase + j * CH, CH)])
            return carry

        lax.fori_loop(0, nch // 2, pair, 0)

    return map_kernel


def _make_tab_kernel(n_pad):
    wpf = NW // F
    per_w = F * n_pad // NW
    nch = per_w // CH
    mesh = plsc.VectorSubcoreMesh(core_axis_name="c", subcore_axis_name="s")

    @functools.partial(
        pl.kernel,
        out_type=jax.ShapeDtypeStruct((F * n_pad, DP), jnp.float32),
        mesh=mesh,
        scratch_types=[
            pltpu.VMEM((per_w,), jnp.int32),
            pltpu.VMEM((2, CH, DP), jnp.float32),
            pltpu.SemaphoreType.DMA,
            pltpu.SemaphoreType.DMA,
        ],
    )
    def tab_kernel(fidx_hbm, tab_hbm, out_hbm, fidx_v, rows_v, sem0, sem1):
        c = lax.axis_index("c")
        si = lax.axis_index("s")
        wid = si * NC + c
        f = wid // wpf
        base = wid * per_w
        tab_f = tab_hbm.at[f]
        sems = (sem0, sem1)
        pltpu.sync_copy(fidx_hbm.at[pl.ds(base, per_w)], fidx_v)
        pltpu.async_copy(
            tab_f.at[fidx_v.at[pl.ds(0, CH)]], rows_v.at[0], sems[0])

        def pair(i, carry):
            for p in range(2):
                k = 2 * i + p
                nxt = (p + 1) % 2

                @pl.when(k + 1 < nch)
                def _():
                    pltpu.async_copy(
                        tab_f.at[fidx_v.at[pl.ds((k + 1) * CH, CH)]],
                        rows_v.at[nxt], sems[nxt])

                pltpu.make_async_copy(
                    tab_f.at[pl.ds(0, CH)], rows_v.at[p], sems[p]).wait()
                pltpu.sync_copy(rows_v.at[p], out_hbm.at[pl.ds(base + k * CH, CH)])
            return carry

        lax.fori_loop(0, nch // 2, pair, 0)

    return tab_kernel


@jax.jit
def kernel(token_seqs, tables, mappings):
    b, s = token_seqs.shape
    sp = (s + 7) // 8 * 8
    n_pad = b * sp
    spread = (lax.broadcasted_iota(jnp.int32, (b, sp - s), 0) * (sp - s)
              + lax.broadcasted_iota(jnp.int32, (b, sp - s), 1)) % V
    tok_flat = jnp.concatenate([token_seqs, spread], axis=1).reshape(n_pad)
    map_flat = mappings.reshape(F * V)
    fidx = _make_map_kernel(n_pad)(tok_flat, map_flat)
    tab128 = jnp.pad(tables, ((0, 0), (0, 0), (0, DP - D)))
    out_k = _make_tab_kernel(n_pad)(fidx, tab128)
    out_tensor = out_k.reshape(F, b, sp, DP)[:, :, :s, :D]
    zero = lax.optimization_barrier(jnp.int32(0))
    out_indices = fidx.reshape(F, b, sp)[:, :, :s] ^ zero
    return (out_tensor, out_indices)


# speedup vs baseline: 1.0042x; 1.0042x over previous
import functools

import jax
import jax.numpy as jnp
from jax import lax
from jax.experimental import pallas as pl
from jax.experimental.pallas import tpu as pltpu
from jax.experimental.pallas import tpu_sc as plsc

F = 4
V = 100002
D = 64
DP = 128
NC = 2
NS = 16
NW = NC * NS
CH = 128


def _make_map_kernel(n_pad):
    wpf = NW // F
    per_w = F * n_pad // NW
    nch = per_w // CH
    mesh = plsc.VectorSubcoreMesh(core_axis_name="c", subcore_axis_name="s")

    @functools.partial(
        pl.kernel,
        out_type=jax.ShapeDtypeStruct((F * n_pad,), jnp.int32),
        mesh=mesh,
        scratch_types=[
            pltpu.VMEM((per_w,), jnp.int32),
            pltpu.VMEM((2, CH), jnp.int32),
            pltpu.VMEM((2, CH), jnp.int32),
            pltpu.SemaphoreType.DMA,
            pltpu.SemaphoreType.DMA,
        ],
    )
    def map_kernel(tok_hbm, map_hbm, fidx_hbm, tok_v, midx_v, val_v, sem0, sem1):
        c = lax.axis_index("c")
        s = lax.axis_index("s")
        wid = s * NC + c
        f = wid // wpf
        foff = f * V
        tbase = (wid - f * wpf) * per_w
        obase = f * n_pad + tbase
        sems = (sem0, sem1)
        pltpu.sync_copy(tok_hbm.at[pl.ds(tbase, per_w)], tok_v)

        def fill(j, buf):
            for i in range(CH // 16):
                midx_v[buf, pl.ds(i * 16, 16)] = (
                    tok_v[pl.ds(j * CH + i * 16, 16)] + foff)

        fill(0, 0)
        pltpu.async_copy(map_hbm.at[midx_v.at[0]], val_v.at[0], sems[0])

        def pair(i, carry):
            for p in range(2):
                j = 2 * i + p
                nxt = (p + 1) % 2

                @pl.when(j + 1 < nch)
                def _():
                    fill(j + 1, nxt)
                    pltpu.async_copy(
                        map_hbm.at[midx_v.at[nxt]], val_v.at[nxt], sems[nxt])

                pltpu.make_async_copy(
                    map_hbm.at[pl.ds(0, CH)], val_v.at[p], sems[p]).wait()
                pltpu.sync_copy(val_v.at[p], fidx_hbm.at[pl.ds(obase + j * CH, CH)])
            return carry

        lax.fori_loop(0, nch // 2, pair, 0)

    return map_kernel


def _make_tab_kernel(n_pad):
    wpf = NW // F
    per_w = F * n_pad // NW
    nch = per_w // CH
    mesh = plsc.VectorSubcoreMesh(core_axis_name="c", subcore_axis_name="s")

    @functools.partial(
        pl.kernel,
        out_type=jax.ShapeDtypeStruct((F * n_pad, DP), jnp.float32),
        mesh=mesh,
        scratch_types=[
            pltpu.VMEM((per_w,), jnp.int32),
            pltpu.VMEM((2, CH, DP), jnp.float32),
            pltpu.SemaphoreType.DMA,
            pltpu.SemaphoreType.DMA,
        ],
    )
    def tab_kernel(fidx_hbm, tab_hbm, out_hbm, fidx_v, rows_v, sem0, sem1):
        c = lax.axis_index("c")
        si = lax.axis_index("s")
        wid = si * NC + c
        f = wid // wpf
        base = wid * per_w
        tab_f = tab_hbm.at[f]
        sems = (sem0, sem1)
        pltpu.sync_copy(fidx_hbm.at[pl.ds(base, per_w)], fidx_v)
        pltpu.async_copy(
            tab_f.at[fidx_v.at[pl.ds(0, CH)]], rows_v.at[0], sems[0])

        def pair(i, carry):
            for p in range(2):
                k = 2 * i + p
                nxt = (p + 1) % 2

                @pl.when(k + 1 < nch)
                def _():
                    pltpu.async_copy(
                        tab_f.at[fidx_v.at[pl.ds((k + 1) * CH, CH)]],
                        rows_v.at[nxt], sems[nxt])

                pltpu.make_async_copy(
                    tab_f.at[pl.ds(0, CH)], rows_v.at[p], sems[p]).wait()
                pltpu.sync_copy(rows_v.at[p], out_hbm.at[pl.ds(base + k * CH, CH)])
            return carry

        lax.fori_loop(0, nch // 2, pair, 0)

    return tab_kernel


@jax.jit
def kernel(token_seqs, tables, mappings):
    b, s = token_seqs.shape
    sp = (s + 7) // 8 * 8
    n_pad = b * sp
    spread = (lax.broadcasted_iota(jnp.int32, (b, sp - s), 0) * (sp - s)
              + lax.broadcasted_iota(jnp.int32, (b, sp - s), 1)) % V
    tok_flat = jnp.concatenate([token_seqs, spread], axis=1).reshape(n_pad)
    map_flat = mappings.reshape(F * V)
    fidx = _make_map_kernel(n_pad)(tok_flat, map_flat)
    tab128 = jnp.pad(tables, ((0, 0), (0, 0), (0, DP - D)))
    out_k = _make_tab_kernel(n_pad)(fidx, tab128)
    out_tensor = out_k.reshape(F, b, sp, DP)[:, :, :s, :D]
    out_indices = fidx.reshape(F, b, sp)[:, :, :s]
    return (out_tensor, out_indices)
